# baseline (speedup 1.0000x reference)
import functools

import jax
import jax.numpy as jnp
from jax import lax
from jax.experimental import pallas as pl
from jax.experimental.pallas import tpu as pltpu
from jax.experimental.pallas import tpu_sc as plsc

B = 16384
D = 128
L = 16
NC = 2
NS = 16
NW = NC * NS
BPW = B // NW
BLK = 64
NBLK = BPW // BLK
DEPTH = 3

_mesh = plsc.VectorSubcoreMesh(core_axis_name="c", subcore_axis_name="s")


@functools.partial(
    pl.kernel,
    mesh=_mesh,
    out_type=jax.ShapeDtypeStruct((B,), jnp.float32),
    compiler_params=pltpu.CompilerParams(needs_layout_passes=False),
    scratch_types=[
        pltpu.VMEM((NBLK, BLK), jnp.int32),
        pltpu.VMEM((NBLK, BLK), jnp.int32),
        pltpu.VMEM((DEPTH, BLK, D), jnp.float32),
        pltpu.VMEM((DEPTH, BLK, D), jnp.float32),
        pltpu.VMEM((BPW,), jnp.float32),
        pltpu.VMEM((BPW,), jnp.float32),
        pltpu.VMEM((L, L + 1), jnp.float32),
        pltpu.SemaphoreType.DMA,
        pltpu.SemaphoreType.DMA,
        pltpu.SemaphoreType.DMA,
    ],
)
def _disc_kernel(nid_hbm, nbr_hbm, emb_hbm, bias_hbm, out_hbm,
                 nid_v, nbr_v, nrows_v, brows_v,
                 bias_v, scores_v, tp_v, sem0, sem1, sem2):
    wid = lax.axis_index("c") * NS + lax.axis_index("s")
    base = wid * NBLK

    sems = (sem0, sem1, sem2)

    cp_i = pltpu.async_copy(nid_hbm.at[pl.ds(base, NBLK)], nid_v, sem0)
    cp_j = pltpu.async_copy(nbr_hbm.at[pl.ds(base, NBLK)], nbr_v, sem1)
    cp_i.wait()
    cp_j.wait()

    def start(blk):
        slot = blk % DEPTH
        sem = sems[slot]
        return (
            pltpu.async_copy(emb_hbm.at[nid_v.at[blk]], nrows_v.at[slot],
                             sem),
            pltpu.async_copy(emb_hbm.at[nbr_v.at[blk]], brows_v.at[slot],
                             sem),
            pltpu.async_copy(bias_hbm.at[nbr_v.at[blk]],
                             bias_v.at[pl.ds(blk * BLK, BLK)], sem),
        )

    lanes = lax.iota(jnp.int32, L)
    inflight = [start(0), start(1)]
    for blk in range(NBLK):
        slot = blk % DEPTH
        if blk + 2 < NBLK:
            inflight.append(start(blk + 2))
        for cp in inflight.pop(0):
            cp.wait()
        nr = nrows_v.at[slot]
        br = brows_v.at[slot]

        def body(g, carry, nr=nr, br=br, blk=blk):
            NI = 4
            for k in range(0, L, NI):
                ps = [g * L + k + i for i in range(NI)]
                accs = [nr[p, pl.ds(0, L)] * br[p, pl.ds(0, L)] for p in ps]
                for c in range(1, D // L):
                    for i, p in enumerate(ps):
                        accs[i] = accs[i] + (nr[p, pl.ds(c * L, L)]
                                             * br[p, pl.ds(c * L, L)])
                for i in range(NI):
                    tp_v[k + i, pl.ds(0, L)] = accs[i]
            g16 = [plsc.load_gather(tp_v, [lanes, jnp.full((L,), c, jnp.int32)])
                   for c in range(L)]
            while len(g16) > 1:
                g16 = [g16[i] + g16[i + 1] for i in range(0, len(g16), 2)]
            scores_v[pl.ds(blk * BLK + g * L, L)] = g16[0]
            return carry

        lax.fori_loop(0, BLK // L, body, 0)

    def sig_body(t, carry):
        for i in range(4):
            off = (t * 4 + i) * L
            s = scores_v[pl.ds(off, L)] + bias_v[pl.ds(off, L)]
            scores_v[pl.ds(off, L)] = 1.0 / (1.0 + jnp.exp(-s))
        return carry

    lax.fori_loop(0, BPW // L // 4, sig_body, 0)

    pltpu.sync_copy(scores_v, out_hbm.at[pl.ds(wid * BPW, BPW)])


def kernel(node_id, node_neighbor_id, embedding_matrix, bias):
    return _disc_kernel(
        node_id.astype(jnp.int32).reshape(B // BLK, BLK),
        node_neighbor_id.astype(jnp.int32).reshape(B // BLK, BLK),
        embedding_matrix,
        bias,
    )


# speedup vs baseline: 1.0263x; 1.0263x over previous
import functools

import jax
import jax.numpy as jnp
from jax import lax
from jax.experimental import pallas as pl
from jax.experimental.pallas import tpu as pltpu
from jax.experimental.pallas import tpu_sc as plsc

B = 16384
D = 128
L = 16
NC = 2
NS = 16
NW = NC * NS
BPW = B // NW
BLK = 128
NBLK = BPW // BLK
DEPTH = 3

_mesh = plsc.VectorSubcoreMesh(core_axis_name="c", subcore_axis_name="s")


@functools.partial(
    pl.kernel,
    mesh=_mesh,
    out_type=jax.ShapeDtypeStruct((B,), jnp.float32),
    compiler_params=pltpu.CompilerParams(needs_layout_passes=False),
    scratch_types=[
        pltpu.VMEM((NBLK, BLK), jnp.int32),
        pltpu.VMEM((NBLK, BLK), jnp.int32),
        pltpu.VMEM((DEPTH, BLK, D), jnp.float32),
        pltpu.VMEM((DEPTH, BLK, D), jnp.float32),
        pltpu.VMEM((BPW,), jnp.float32),
        pltpu.VMEM((BPW,), jnp.float32),
        pltpu.VMEM((L, L + 1), jnp.float32),
        pltpu.SemaphoreType.DMA,
        pltpu.SemaphoreType.DMA,
        pltpu.SemaphoreType.DMA,
    ],
)
def _disc_kernel(nid_hbm, nbr_hbm, emb_hbm, bias_hbm, out_hbm,
                 nid_v, nbr_v, nrows_v, brows_v,
                 bias_v, scores_v, tp_v, sem0, sem1, sem2):
    wid = lax.axis_index("c") * NS + lax.axis_index("s")
    base = wid * NBLK

    sems = (sem0, sem1, sem2)

    cp_i = pltpu.async_copy(nid_hbm.at[pl.ds(base, NBLK)], nid_v, sem0)
    cp_j = pltpu.async_copy(nbr_hbm.at[pl.ds(base, NBLK)], nbr_v, sem1)
    cp_i.wait()
    cp_j.wait()

    def start(blk):
        slot = blk % DEPTH
        sem = sems[slot]
        return (
            pltpu.async_copy(emb_hbm.at[nid_v.at[blk]], nrows_v.at[slot],
                             sem),
            pltpu.async_copy(emb_hbm.at[nbr_v.at[blk]], brows_v.at[slot],
                             sem),
            pltpu.async_copy(bias_hbm.at[nbr_v.at[blk]],
                             bias_v.at[pl.ds(blk * BLK, BLK)], sem),
        )

    lanes = lax.iota(jnp.int32, L)
    inflight = [start(0), start(1)]
    for blk in range(NBLK):
        slot = blk % DEPTH
        if blk + 2 < NBLK:
            inflight.append(start(blk + 2))
        for cp in inflight.pop(0):
            cp.wait()
        nr = nrows_v.at[slot]
        br = brows_v.at[slot]

        def body(g, carry, nr=nr, br=br, blk=blk):
            NI = 4
            for k in range(0, L, NI):
                ps = [g * L + k + i for i in range(NI)]
                accs = [nr[p, pl.ds(0, L)] * br[p, pl.ds(0, L)] for p in ps]
                for c in range(1, D // L):
                    for i, p in enumerate(ps):
                        accs[i] = accs[i] + (nr[p, pl.ds(c * L, L)]
                                             * br[p, pl.ds(c * L, L)])
                for i in range(NI):
                    tp_v[k + i, pl.ds(0, L)] = accs[i]
            g16 = [plsc.load_gather(tp_v, [lanes, jnp.full((L,), c, jnp.int32)])
                   for c in range(L)]
            while len(g16) > 1:
                g16 = [g16[i] + g16[i + 1] for i in range(0, len(g16), 2)]
            scores_v[pl.ds(blk * BLK + g * L, L)] = g16[0]
            return carry

        lax.fori_loop(0, BLK // L, body, 0)

    def sig_body(t, carry):
        for i in range(4):
            off = (t * 4 + i) * L
            s = scores_v[pl.ds(off, L)] + bias_v[pl.ds(off, L)]
            scores_v[pl.ds(off, L)] = 1.0 / (1.0 + jnp.exp(-s))
        return carry

    lax.fori_loop(0, BPW // L // 4, sig_body, 0)

    pltpu.sync_copy(scores_v, out_hbm.at[pl.ds(wid * BPW, BPW)])


def kernel(node_id, node_neighbor_id, embedding_matrix, bias):
    return _disc_kernel(
        node_id.astype(jnp.int32).reshape(B // BLK, BLK),
        node_neighbor_id.astype(jnp.int32).reshape(B // BLK, BLK),
        embedding_matrix,
        bias,
    )


# speedup vs baseline: 1.0531x; 1.0261x over previous
import functools

import jax
import jax.numpy as jnp
from jax import lax
from jax.experimental import pallas as pl
from jax.experimental.pallas import tpu as pltpu
from jax.experimental.pallas import tpu_sc as plsc

B = 16384
D = 128
L = 16
NC = 2
NS = 16
NW = NC * NS
BPW = B // NW
BLK = 128
NBLK = BPW // BLK
H = BLK // 2

_mesh = plsc.VectorSubcoreMesh(core_axis_name="c", subcore_axis_name="s")


@functools.partial(
    pl.kernel,
    mesh=_mesh,
    out_type=jax.ShapeDtypeStruct((B,), jnp.float32),
    compiler_params=pltpu.CompilerParams(needs_layout_passes=False),
    scratch_types=[
        pltpu.VMEM((NBLK, BLK), jnp.int32),
        pltpu.VMEM((NBLK, BLK), jnp.int32),
        pltpu.VMEM((BLK, D), jnp.float32),
        pltpu.VMEM((BLK, D), jnp.float32),
        pltpu.VMEM((BLK, D), jnp.float32),
        pltpu.VMEM((BLK, D), jnp.float32),
        pltpu.VMEM((BPW,), jnp.float32),
        pltpu.VMEM((BPW,), jnp.float32),
        pltpu.VMEM((L, L + 1), jnp.float32),
        pltpu.SemaphoreType.DMA,
        pltpu.SemaphoreType.DMA,
        pltpu.SemaphoreType.DMA,
        pltpu.SemaphoreType.DMA,
    ],
)
def _disc_kernel(nid_hbm, nbr_hbm, emb_hbm, bias_hbm, out_hbm,
                 nid_v, nbr_v, nrows0_v, nrows1_v, brows0_v, brows1_v,
                 bias_v, scores_v, tp_v, sem0, sem1, semh, semz):
    wid = lax.axis_index("c") * NS + lax.axis_index("s")
    base = wid * NBLK

    nrows = (nrows0_v, nrows1_v)
    brows = (brows0_v, brows1_v)
    sems = (sem0, sem1)

    cp_i = pltpu.async_copy(nid_hbm.at[pl.ds(base, NBLK)], nid_v, sem0)
    cp_j = pltpu.async_copy(nbr_hbm.at[pl.ds(base, NBLK)], nbr_v, sem1)
    cp_i.wait()
    cp_j.wait()

    s0a = (
        pltpu.async_copy(emb_hbm.at[nid_v.at[0, pl.ds(0, H)]],
                         nrows0_v.at[pl.ds(0, H)], sem0),
        pltpu.async_copy(emb_hbm.at[nbr_v.at[0, pl.ds(0, H)]],
                         brows0_v.at[pl.ds(0, H)], sem0),
    )
    s0b = (
        pltpu.async_copy(emb_hbm.at[nid_v.at[0, pl.ds(H, H)]],
                         nrows0_v.at[pl.ds(H, H)], semh),
        pltpu.async_copy(emb_hbm.at[nbr_v.at[0, pl.ds(H, H)]],
                         brows0_v.at[pl.ds(H, H)], semh),
    )

    def start(blk):
        slot = blk % 2
        sem = sems[slot]
        return (
            pltpu.async_copy(emb_hbm.at[nid_v.at[blk]], nrows[slot], sem),
            pltpu.async_copy(emb_hbm.at[nbr_v.at[blk]], brows[slot], sem),
        )

    def start_bias(blk):
        return pltpu.async_copy(bias_hbm.at[nbr_v.at[blk]],
                                bias_v.at[pl.ds(blk * BLK, BLK)], semz)

    lanes = lax.iota(jnp.int32, L)

    def make_body(nr, br, blk, g0):
        def body(g, carry):
            NI = 4
            for k in range(0, L, NI):
                ps = [(g + g0) * L + k + i for i in range(NI)]
                accs = [nr[p, pl.ds(0, L)] * br[p, pl.ds(0, L)] for p in ps]
                for c in range(1, D // L):
                    for i, p in enumerate(ps):
                        accs[i] = accs[i] + (nr[p, pl.ds(c * L, L)]
                                             * br[p, pl.ds(c * L, L)])
                for i in range(NI):
                    tp_v[k + i, pl.ds(0, L)] = accs[i]
            g16 = [plsc.load_gather(tp_v, [lanes, jnp.full((L,), c, jnp.int32)])
                   for c in range(L)]
            while len(g16) > 1:
                g16 = [g16[i] + g16[i + 1] for i in range(0, len(g16), 2)]
            scores_v[pl.ds(blk * BLK + (g + g0) * L, L)] = g16[0]
            return carry
        return body

    inflight = start(1)
    bias_cps = [start_bias(0), start_bias(1)]
    for cp in s0a:
        cp.wait()
    lax.fori_loop(0, H // L, make_body(nrows0_v, brows0_v, 0, 0), 0)
    for cp in s0b:
        cp.wait()
    lax.fori_loop(0, H // L, make_body(nrows0_v, brows0_v, 0, H // L), 0)

    for blk in range(1, NBLK):
        slot = blk % 2
        nr, br = nrows[slot], brows[slot]
        if blk + 1 < NBLK:
            nxt = start(blk + 1)
            bias_cps.append(start_bias(blk + 1))
        else:
            nxt = None
        for cp in inflight:
            cp.wait()
        inflight = nxt
        lax.fori_loop(0, BLK // L, make_body(nr, br, blk, 0), 0)

    for cp in bias_cps:
        cp.wait()

    def sig_body(t, carry):
        for i in range(4):
            off = (t * 4 + i) * L
            s = scores_v[pl.ds(off, L)] + bias_v[pl.ds(off, L)]
            scores_v[pl.ds(off, L)] = 1.0 / (1.0 + jnp.exp(-s))
        return carry

    lax.fori_loop(0, BPW // L // 4, sig_body, 0)

    pltpu.sync_copy(scores_v, out_hbm.at[pl.ds(wid * BPW, BPW)])


def kernel(node_id, node_neighbor_id, embedding_matrix, bias):
    return _disc_kernel(
        node_id.astype(jnp.int32).reshape(B // BLK, BLK),
        node_neighbor_id.astype(jnp.int32).reshape(B // BLK, BLK),
        embedding_matrix,
        bias,
    )


# speedup vs baseline: 1.0941x; 1.0389x over previous
import functools

import jax
import jax.numpy as jnp
from jax import lax
from jax.experimental import pallas as pl
from jax.experimental.pallas import tpu as pltpu
from jax.experimental.pallas import tpu_sc as plsc

B = 16384
D = 128
L = 16
NC = 2
NS = 16
NW = NC * NS
BPW = B // NW
BLK = 128
NBLK = BPW // BLK

_mesh = plsc.VectorSubcoreMesh(core_axis_name="c", subcore_axis_name="s")


@functools.partial(
    pl.kernel,
    mesh=_mesh,
    out_type=jax.ShapeDtypeStruct((B,), jnp.float32),
    compiler_params=pltpu.CompilerParams(needs_layout_passes=False),
    scratch_types=[
        pltpu.VMEM((NBLK, BLK), jnp.int32),
        pltpu.VMEM((NBLK, BLK), jnp.int32),
        pltpu.VMEM((2 * BLK, D), jnp.float32),
        pltpu.VMEM((2 * BLK, D), jnp.float32),
        pltpu.VMEM((BPW,), jnp.float32),
        pltpu.VMEM((BPW,), jnp.float32),
        pltpu.VMEM((L, L + 1), jnp.float32),
        pltpu.SemaphoreType.DMA,
        pltpu.SemaphoreType.DMA,
        pltpu.SemaphoreType.DMA,
    ],
)
def _disc_kernel(nid_hbm, nbr_hbm, emb_hbm, bias_hbm, out_hbm,
                 nid_v, nbr_v, nrows_v, brows_v,
                 bias_v, scores_v, tp_v, sem0, sem1, semz):
    wid = lax.axis_index("c") * NS + lax.axis_index("s")
    base = wid * NBLK

    cp_i = pltpu.async_copy(nid_hbm.at[pl.ds(base, NBLK)], nid_v, sem0)
    cp_j = pltpu.async_copy(nbr_hbm.at[pl.ds(base, NBLK)], nbr_v, sem1)
    cp_i.wait()
    cp_j.wait()

    def start_rows(blk, slot0):
        off = 0 if slot0 else BLK
        sem = sem0 if slot0 else sem1
        pltpu.async_copy(emb_hbm.at[nid_v.at[blk]],
                         nrows_v.at[pl.ds(off, BLK)], sem)
        pltpu.async_copy(emb_hbm.at[nbr_v.at[blk]],
                         brows_v.at[pl.ds(off, BLK)], sem)

    def wait_rows(slot0):
        off = 0 if slot0 else BLK
        sem = sem0 if slot0 else sem1
        pltpu.make_async_copy(emb_hbm.at[pl.ds(0, BLK)],
                              nrows_v.at[pl.ds(off, BLK)], sem).wait()
        pltpu.make_async_copy(emb_hbm.at[pl.ds(0, BLK)],
                              brows_v.at[pl.ds(off, BLK)], sem).wait()

    for blk in range(NBLK):
        pltpu.async_copy(bias_hbm.at[nbr_v.at[blk]],
                         bias_v.at[pl.ds(blk * BLK, BLK)], semz)

    start_rows(0, True)
    start_rows(1, False)

    lanes = lax.iota(jnp.int32, L)

    def block_body(t, carry):
        even = t % 2 == 0
        soff = jnp.where(even, 0, BLK)

        @pl.when(jnp.logical_and(t + 2 < NBLK, even))
        def _():
            start_rows(t + 2, True)

        @pl.when(jnp.logical_and(t + 2 < NBLK, jnp.logical_not(even)))
        def _():
            start_rows(t + 2, False)

        @pl.when(even)
        def _():
            wait_rows(True)

        @pl.when(jnp.logical_not(even))
        def _():
            wait_rows(False)

        def body(g, carry2):
            NI = 4
            for k in range(0, L, NI):
                ps = [soff + g * L + k + i for i in range(NI)]
                accs = [nrows_v[p, pl.ds(0, L)] * brows_v[p, pl.ds(0, L)]
                        for p in ps]
                for c in range(1, D // L):
                    for i, p in enumerate(ps):
                        accs[i] = accs[i] + (nrows_v[p, pl.ds(c * L, L)]
                                             * brows_v[p, pl.ds(c * L, L)])
                for i in range(NI):
                    tp_v[k + i, pl.ds(0, L)] = accs[i]
            g16 = [plsc.load_gather(tp_v, [lanes, jnp.full((L,), c, jnp.int32)])
                   for c in range(L)]
            while len(g16) > 1:
                g16 = [g16[i] + g16[i + 1] for i in range(0, len(g16), 2)]
            scores_v[pl.ds(t * BLK + g * L, L)] = g16[0]
            return carry2

        lax.fori_loop(0, BLK // L, body, 0)
        return carry

    lax.fori_loop(0, NBLK, block_body, 0)

    for blk in range(NBLK):
        pltpu.make_async_copy(bias_hbm.at[pl.ds(0, BLK)],
                              bias_v.at[pl.ds(blk * BLK, BLK)], semz).wait()

    def sig_body(t, carry):
        for i in range(4):
            off = (t * 4 + i) * L
            s = scores_v[pl.ds(off, L)] + bias_v[pl.ds(off, L)]
            scores_v[pl.ds(off, L)] = 1.0 / (1.0 + jnp.exp(-s))
        return carry

    lax.fori_loop(0, BPW // L // 4, sig_body, 0)

    pltpu.sync_copy(scores_v, out_hbm.at[pl.ds(wid * BPW, BPW)])


def kernel(node_id, node_neighbor_id, embedding_matrix, bias):
    return _disc_kernel(
        node_id.astype(jnp.int32).reshape(B // BLK, BLK),
        node_neighbor_id.astype(jnp.int32).reshape(B // BLK, BLK),
        embedding_matrix,
        bias,
    )


# speedup vs baseline: 1.1301x; 1.0329x over previous
import functools

import jax
import jax.numpy as jnp
from jax import lax
from jax.experimental import pallas as pl
from jax.experimental.pallas import tpu as pltpu
from jax.experimental.pallas import tpu_sc as plsc

B = 16384
D = 128
L = 16
NC = 2
NS = 16
NW = NC * NS
BPW = B // NW
BLK = 128
NBLK = BPW // BLK

_mesh = plsc.VectorSubcoreMesh(core_axis_name="c", subcore_axis_name="s")


@functools.partial(
    pl.kernel,
    mesh=_mesh,
    out_type=jax.ShapeDtypeStruct((B,), jnp.float32),
    compiler_params=pltpu.CompilerParams(needs_layout_passes=False),
    scratch_types=[
        pltpu.VMEM((NBLK, BLK), jnp.int32),
        pltpu.VMEM((NBLK, BLK), jnp.int32),
        pltpu.VMEM((2 * BLK, D), jnp.float32),
        pltpu.VMEM((2 * BLK, D), jnp.float32),
        pltpu.VMEM((BPW,), jnp.float32),
        pltpu.VMEM((BPW,), jnp.float32),
        pltpu.VMEM((L, L + 1), jnp.float32),
        pltpu.SemaphoreType.DMA,
        pltpu.SemaphoreType.DMA,
        pltpu.SemaphoreType.DMA,
    ],
)
def _disc_kernel(nid_hbm, nbr_hbm, emb_hbm, bias_hbm, out_hbm,
                 nid_v, nbr_v, nrows_v, brows_v,
                 bias_v, scores_v, tp_v, sem0, sem1, semz):
    wid = lax.axis_index("c") * NS + lax.axis_index("s")
    base = wid * NBLK

    cp_i = pltpu.async_copy(nid_hbm.at[pl.ds(base, NBLK)], nid_v, sem0)
    cp_j = pltpu.async_copy(nbr_hbm.at[pl.ds(base, NBLK)], nbr_v, sem1)
    cp_i.wait()
    cp_j.wait()

    def start_rows(blk, slot0):
        off = 0 if slot0 else BLK
        sem = sem0 if slot0 else sem1
        pltpu.async_copy(emb_hbm.at[nid_v.at[blk]],
                         nrows_v.at[pl.ds(off, BLK)], sem)
        pltpu.async_copy(emb_hbm.at[nbr_v.at[blk]],
                         brows_v.at[pl.ds(off, BLK)], sem)

    def wait_rows(slot0):
        off = 0 if slot0 else BLK
        sem = sem0 if slot0 else sem1
        pltpu.make_async_copy(emb_hbm.at[pl.ds(0, BLK)],
                              nrows_v.at[pl.ds(off, BLK)], sem).wait()
        pltpu.make_async_copy(emb_hbm.at[pl.ds(0, BLK)],
                              brows_v.at[pl.ds(off, BLK)], sem).wait()

    for blk in range(NBLK):
        pltpu.async_copy(bias_hbm.at[nbr_v.at[blk]],
                         bias_v.at[pl.ds(blk * BLK, BLK)], semz)

    start_rows(0, True)

    lanes = lax.iota(jnp.int32, L)

    def block_body(t, carry):
        even = t % 2 == 0
        soff = jnp.where(even, 0, BLK)

        @pl.when(jnp.logical_and(t + 1 < NBLK, even))
        def _():
            start_rows(t + 1, False)

        @pl.when(jnp.logical_and(t + 1 < NBLK, jnp.logical_not(even)))
        def _():
            start_rows(t + 1, True)

        @pl.when(even)
        def _():
            wait_rows(True)

        @pl.when(jnp.logical_not(even))
        def _():
            wait_rows(False)

        def body(g, carry2):
            NI = 4
            for k in range(0, L, NI):
                ps = [soff + g * L + k + i for i in range(NI)]
                accs = [nrows_v[p, pl.ds(0, L)] * brows_v[p, pl.ds(0, L)]
                        for p in ps]
                for c in range(1, D // L):
                    for i, p in enumerate(ps):
                        accs[i] = accs[i] + (nrows_v[p, pl.ds(c * L, L)]
                                             * brows_v[p, pl.ds(c * L, L)])
                for i in range(NI):
                    tp_v[k + i, pl.ds(0, L)] = accs[i]
            g16 = [plsc.load_gather(tp_v, [lanes, jnp.full((L,), c, jnp.int32)])
                   for c in range(L)]
            while len(g16) > 1:
                g16 = [g16[i] + g16[i + 1] for i in range(0, len(g16), 2)]
            scores_v[pl.ds(t * BLK + g * L, L)] = g16[0]
            return carry2

        lax.fori_loop(0, BLK // L, body, 0)
        return carry

    lax.fori_loop(0, NBLK, block_body, 0)

    for blk in range(NBLK):
        pltpu.make_async_copy(bias_hbm.at[pl.ds(0, BLK)],
                              bias_v.at[pl.ds(blk * BLK, BLK)], semz).wait()

    def sig_body(t, carry):
        for i in range(4):
            off = (t * 4 + i) * L
            s = scores_v[pl.ds(off, L)] + bias_v[pl.ds(off, L)]
            scores_v[pl.ds(off, L)] = 1.0 / (1.0 + jnp.exp(-s))
        return carry

    lax.fori_loop(0, BPW // L // 4, sig_body, 0)

    pltpu.sync_copy(scores_v, out_hbm.at[pl.ds(wid * BPW, BPW)])


def kernel(node_id, node_neighbor_id, embedding_matrix, bias):
    return _disc_kernel(
        node_id.astype(jnp.int32).reshape(B // BLK, BLK),
        node_neighbor_id.astype(jnp.int32).reshape(B // BLK, BLK),
        embedding_matrix,
        bias,
    )
